# baseline (speedup 1.0000x reference)
import functools

import jax
import jax.numpy as jnp
from jax.experimental import pallas as pl
from jax.experimental.pallas import tpu as pltpu

DURATION = 100.0


def _fused_body(x_ref, off_ref, out_ref, acc_ref, occ_ref):
    i = pl.program_id(0)
    nsteps = pl.num_programs(0)
    nclass = off_ref.shape[1]

    off = off_ref[...]
    norms = jnp.sum(jnp.abs(off), axis=1, keepdims=True)
    prop = off / jnp.maximum(norms, 1e-12)
    maxv = jnp.max(prop, axis=1, keepdims=True)
    iota = jax.lax.broadcasted_iota(jnp.int32, prop.shape, 1)
    is_max = prop == maxv
    amax = jnp.min(jnp.where(is_max, iota, nclass), axis=1, keepdims=True)
    oh = iota == amax
    assoc = jnp.where(oh, prop, 0.0)

    @pl.when(i == 0)
    def _init():
        acc_ref[...] = jnp.zeros_like(acc_ref)
        occ_ref[...] = jnp.zeros_like(occ_ref)

    occ_ref[...] += jnp.sum(oh.astype(jnp.float32), axis=0, keepdims=True)
    x = (DURATION - x_ref[...]) * (1.0 / DURATION)
    acc_ref[...] += jnp.dot(x, assoc, preferred_element_type=jnp.float32)

    @pl.when(i == nsteps - 1)
    def _fini():
        occ = jnp.maximum(occ_ref[...], 1.0)
        out_ref[...] = acc_ref[...] / occ


def kernel(inputs, offsets):
    batch, nneuron = inputs.shape
    nclass = offsets.shape[1]
    blk_n = 2048
    grid = nneuron // blk_n
    return pl.pallas_call(
        _fused_body,
        grid=(grid,),
        in_specs=[
            pl.BlockSpec((batch, blk_n), lambda i: (0, i)),
            pl.BlockSpec((blk_n, nclass), lambda i: (i, 0)),
        ],
        out_specs=pl.BlockSpec((batch, nclass), lambda i: (0, 0)),
        out_shape=jax.ShapeDtypeStruct((batch, nclass), jnp.float32),
        scratch_shapes=[
            pltpu.VMEM((batch, nclass), jnp.float32),
            pltpu.VMEM((1, nclass), jnp.float32),
        ],
        compiler_params=pltpu.CompilerParams(
            dimension_semantics=("arbitrary",),
        ),
    )(inputs, offsets)


# speedup vs baseline: 1.0560x; 1.0560x over previous
import functools

import jax
import jax.numpy as jnp
from jax.experimental import pallas as pl
from jax.experimental.pallas import tpu as pltpu

DURATION = 100.0


def _fused_body(x_ref, off_ref, out_ref, acc_ref, occ_ref):
    i = pl.program_id(0)
    nsteps = pl.num_programs(0)
    nclass = off_ref.shape[1]

    off = off_ref[...]
    norms = jnp.sum(jnp.abs(off), axis=1, keepdims=True)
    prop = off / jnp.maximum(norms, 1e-12)
    maxv = jnp.max(prop, axis=1, keepdims=True)
    iota = jax.lax.broadcasted_iota(jnp.int32, prop.shape, 1)
    is_max = prop == maxv
    amax = jnp.min(jnp.where(is_max, iota, nclass), axis=1, keepdims=True)
    oh = iota == amax
    assoc = jnp.where(oh, prop, 0.0)

    @pl.when(i == 0)
    def _init():
        acc_ref[...] = jnp.zeros_like(acc_ref)
        occ_ref[...] = jnp.zeros_like(occ_ref)

    occ_ref[...] += jnp.sum(oh.astype(jnp.float32), axis=0, keepdims=True)
    x = ((DURATION - x_ref[...]) * (1.0 / DURATION)).astype(jnp.bfloat16)
    acc_ref[...] += jnp.dot(
        x, assoc.astype(jnp.bfloat16), preferred_element_type=jnp.float32
    )

    @pl.when(i == nsteps - 1)
    def _fini():
        occ = jnp.maximum(occ_ref[...], 1.0)
        out_ref[...] = acc_ref[...] / occ


def kernel(inputs, offsets):
    batch, nneuron = inputs.shape
    nclass = offsets.shape[1]
    blk_n = 4096
    grid = nneuron // blk_n
    return pl.pallas_call(
        _fused_body,
        grid=(grid,),
        in_specs=[
            pl.BlockSpec((batch, blk_n), lambda i: (0, i)),
            pl.BlockSpec((blk_n, nclass), lambda i: (i, 0)),
        ],
        out_specs=pl.BlockSpec((batch, nclass), lambda i: (0, 0)),
        out_shape=jax.ShapeDtypeStruct((batch, nclass), jnp.float32),
        scratch_shapes=[
            pltpu.VMEM((batch, nclass), jnp.float32),
            pltpu.VMEM((1, nclass), jnp.float32),
        ],
        compiler_params=pltpu.CompilerParams(
            dimension_semantics=("arbitrary",),
        ),
    )(inputs, offsets)
